# Initial kernel scaffold; baseline (speedup 1.0000x reference)
import jax
import jax.numpy as jnp
from jax.experimental import pallas as pl


def kernel(input_ids, bbox, word_emb, pos_emb, x_emb, y_emb, h_emb, w_emb, tt_emb, ln_w, ln_b):
    raise NotImplementedError("write your pallas kernel here")


# speedup vs baseline: 2.2612x; 2.2612x over previous
import functools

import jax
import jax.numpy as jnp
from jax import lax
from jax.experimental import pallas as pl
from jax.experimental.pallas import tpu as pltpu
from jax.experimental.pallas import tpu_sc as plsc

HIDDEN = 768
EPS = 1e-12
SEQ = 512
CHUNK = 32
NSL = HIDDEN // 16


def _sc_gather_sum(ids, x0, y1, x2, y3, word_emb, x_emb, y_emb, h_emb,
                   w_emb, pos_tt):
    n_tok = ids.shape[0]
    info = plsc.get_sparse_core_info()
    n_workers = info.num_cores * info.num_subcores
    per_w = n_tok // n_workers
    n_steps = per_w // CHUNK

    mesh = plsc.VectorSubcoreMesh(core_axis_name="c", subcore_axis_name="s")

    @functools.partial(
        pl.kernel,
        mesh=mesh,
        out_type=jax.ShapeDtypeStruct((n_tok, HIDDEN), jnp.float32),
        scratch_types=[
            pltpu.VMEM((per_w,), jnp.int32),
            pltpu.VMEM((per_w,), jnp.int32),
            pltpu.VMEM((per_w,), jnp.int32),
            pltpu.VMEM((per_w,), jnp.int32),
            pltpu.VMEM((per_w,), jnp.int32),
            pltpu.VMEM((per_w,), jnp.int32),
            pltpu.VMEM((per_w,), jnp.int32),
            pltpu.VMEM((CHUNK, HIDDEN), jnp.float32),
            pltpu.VMEM((CHUNK, HIDDEN), jnp.float32),
            pltpu.VMEM((CHUNK, HIDDEN), jnp.float32),
            pltpu.SemaphoreType.DMA,
            pltpu.SemaphoreType.DMA,
        ],
    )
    def gather_sum(ids_h, x0_h, y1_h, x2_h, y3_h, word_h, x_h, y_h, h_h,
                   w_h, pos_h, out_h, ids_v, x0_v, y1_v, x2_v, y3_v, hx_v,
                   wx_v, acc, gb0, gb1, sem0, sem1):
        wid = lax.axis_index("s") * info.num_cores + lax.axis_index("c")
        base = wid * per_w

        pltpu.sync_copy(ids_h.at[pl.ds(base, per_w)], ids_v)
        pltpu.sync_copy(x0_h.at[pl.ds(base, per_w)], x0_v)
        pltpu.sync_copy(y1_h.at[pl.ds(base, per_w)], y1_v)
        pltpu.sync_copy(x2_h.at[pl.ds(base, per_w)], x2_v)
        pltpu.sync_copy(y3_h.at[pl.ds(base, per_w)], y3_v)

        def hw_body(j, _):
            s = pl.ds(j * 16, 16)
            hx_v[s] = y3_v[s] - y1_v[s]
            wx_v[s] = x2_v[s] - x0_v[s]
            return 0

        lax.fori_loop(0, per_w // 16, hw_body, 0)

        gbufs = (gb0, gb1)
        sems = (sem0, sem1)

        def accum(buf):
            def rbody(r, _):
                for c in range(NSL):
                    cs = pl.ds(c * 16, 16)
                    plsc.addupdate(acc.at[r, cs], buf[r, cs])
                return 0

            lax.fori_loop(0, CHUNK, rbody, 0)

        def step(i, _):
            tok0 = pl.multiple_of(i * CHUNK, CHUNK)
            p0 = pl.multiple_of(lax.rem(i * CHUNK, SEQ), CHUNK)
            pltpu.sync_copy(pos_h.at[pl.ds(p0, CHUNK)], acc)
            sl = pl.ds(tok0, CHUNK)
            tables = (
                (word_h, ids_v),
                (x_h, x0_v),
                (x_h, x2_v),
                (y_h, y1_v),
                (y_h, y3_v),
                (h_h, hx_v),
                (w_h, wx_v),
            )
            cps = [None] * len(tables)
            tbl0, iv0 = tables[0]
            cps[0] = pltpu.async_copy(tbl0.at[iv0.at[sl]], gbufs[0], sems[0])
            for k in range(len(tables)):
                cps[k].wait()
                if k + 1 < len(tables):
                    tbl, iv = tables[k + 1]
                    cps[k + 1] = pltpu.async_copy(
                        tbl.at[iv.at[sl]], gbufs[(k + 1) % 2],
                        sems[(k + 1) % 2])
                accum(gbufs[k % 2])
            dst0 = pl.multiple_of(base + tok0, CHUNK)
            pltpu.sync_copy(acc, out_h.at[pl.ds(dst0, CHUNK)])
            return 0

        lax.fori_loop(0, n_steps, step, 0)

    return gather_sum(ids, x0, y1, x2, y3, word_emb, x_emb, y_emb, h_emb,
                      w_emb, pos_tt)


def _ln_body(x_ref, w_ref, b_ref, o_ref):
    x = x_ref[...]
    mean = jnp.mean(x, axis=-1, keepdims=True)
    xc = x - mean
    var = jnp.mean(xc * xc, axis=-1, keepdims=True)
    o_ref[...] = xc * lax.rsqrt(var + EPS) * w_ref[...] + b_ref[...]


def _layer_norm_tc(x, w, b):
    n_tok = x.shape[0]
    rows = 512
    grid = (n_tok // rows,)
    return pl.pallas_call(
        _ln_body,
        grid=grid,
        in_specs=[
            pl.BlockSpec((rows, HIDDEN), lambda i: (i, 0)),
            pl.BlockSpec((1, HIDDEN), lambda i: (0, 0)),
            pl.BlockSpec((1, HIDDEN), lambda i: (0, 0)),
        ],
        out_specs=pl.BlockSpec((rows, HIDDEN), lambda i: (i, 0)),
        out_shape=jax.ShapeDtypeStruct((n_tok, HIDDEN), jnp.float32),
    )(x, w.reshape(1, HIDDEN), b.reshape(1, HIDDEN))


def kernel(input_ids, bbox, word_emb, pos_emb, x_emb, y_emb, h_emb, w_emb,
           tt_emb, ln_w, ln_b):
    batch, seq = input_ids.shape
    n_tok = batch * seq
    ids = input_ids.reshape(n_tok).astype(jnp.int32)
    bb = bbox.reshape(n_tok, 4).astype(jnp.int32)
    x0, y1, x2, y3 = bb[:, 0], bb[:, 1], bb[:, 2], bb[:, 3]
    pos_tt = pos_emb + tt_emb[0][None, :]
    summed = _sc_gather_sum(ids, x0, y1, x2, y3, word_emb, x_emb, y_emb,
                            h_emb, w_emb, pos_tt)
    out = _layer_norm_tc(summed, ln_w, ln_b)
    return out.reshape(batch, seq, HIDDEN)


# speedup vs baseline: 2.2668x; 1.0024x over previous
import functools

import jax
import jax.numpy as jnp
from jax import lax
from jax.experimental import pallas as pl
from jax.experimental.pallas import tpu as pltpu
from jax.experimental.pallas import tpu_sc as plsc

HIDDEN = 768
EPS = 1e-12
SEQ = 512
CHUNK = 32
NSL = HIDDEN // 16


def _sc_gather_sum(ids, x0, y1, x2, y3, word_emb, x_emb, y_emb, h_emb,
                   w_emb, pos_tt):
    n_tok = ids.shape[0]
    info = plsc.get_sparse_core_info()
    n_workers = info.num_cores * info.num_subcores
    per_w = n_tok // n_workers
    n_steps = per_w // CHUNK

    mesh = plsc.VectorSubcoreMesh(core_axis_name="c", subcore_axis_name="s")

    @functools.partial(
        pl.kernel,
        mesh=mesh,
        out_type=jax.ShapeDtypeStruct((n_tok, HIDDEN), jnp.float32),
        scratch_types=[
            pltpu.VMEM((per_w,), jnp.int32),
            pltpu.VMEM((per_w,), jnp.int32),
            pltpu.VMEM((per_w,), jnp.int32),
            pltpu.VMEM((per_w,), jnp.int32),
            pltpu.VMEM((per_w,), jnp.int32),
            pltpu.VMEM((per_w,), jnp.int32),
            pltpu.VMEM((per_w,), jnp.int32),
            pltpu.VMEM((CHUNK, HIDDEN), jnp.float32),
            pltpu.VMEM((CHUNK, HIDDEN), jnp.float32),
            pltpu.VMEM((CHUNK, HIDDEN), jnp.float32),
            pltpu.VMEM((CHUNK, HIDDEN), jnp.float32),
            pltpu.SemaphoreType.DMA,
            pltpu.SemaphoreType.DMA,
            pltpu.SemaphoreType.DMA,
            pltpu.SemaphoreType.DMA,
        ],
    )
    def gather_sum(ids_h, x0_h, y1_h, x2_h, y3_h, word_h, x_h, y_h, h_h,
                   w_h, pos_h, out_h, ids_v, x0_v, y1_v, x2_v, y3_v, hx_v,
                   wx_v, acc, gb0, gb1, gb2, sem0, sem1, sem2, psem):
        wid = lax.axis_index("s") * info.num_cores + lax.axis_index("c")
        base = wid * per_w

        pltpu.sync_copy(ids_h.at[pl.ds(base, per_w)], ids_v)
        pltpu.sync_copy(x0_h.at[pl.ds(base, per_w)], x0_v)
        pltpu.sync_copy(y1_h.at[pl.ds(base, per_w)], y1_v)
        pltpu.sync_copy(x2_h.at[pl.ds(base, per_w)], x2_v)
        pltpu.sync_copy(y3_h.at[pl.ds(base, per_w)], y3_v)

        def hw_body(j, _):
            s = pl.ds(j * 16, 16)
            hx_v[s] = y3_v[s] - y1_v[s]
            wx_v[s] = x2_v[s] - x0_v[s]
            return 0

        lax.fori_loop(0, per_w // 16, hw_body, 0)

        gbufs = (gb0, gb1, gb2)
        sems = (sem0, sem1, sem2)
        nbuf = 3

        def accum(buf):
            def rbody(r, _):
                for c in range(NSL):
                    cs = pl.ds(c * 16, 16)
                    plsc.addupdate(acc.at[r, cs], buf[r, cs])
                return 0

            lax.fori_loop(0, CHUNK, rbody, 0)

        def step(i, _):
            tok0 = pl.multiple_of(i * CHUNK, CHUNK)
            p0 = pl.multiple_of(lax.rem(i * CHUNK, SEQ), CHUNK)
            cp_pos = pltpu.async_copy(pos_h.at[pl.ds(p0, CHUNK)], acc, psem)
            sl = pl.ds(tok0, CHUNK)
            tables = (
                (word_h, ids_v),
                (x_h, x0_v),
                (x_h, x2_v),
                (y_h, y1_v),
                (y_h, y3_v),
                (h_h, hx_v),
                (w_h, wx_v),
            )
            nt = len(tables)
            cps = [None] * nt
            for k in range(nbuf):
                tbl, iv = tables[k]
                cps[k] = pltpu.async_copy(tbl.at[iv.at[sl]], gbufs[k],
                                          sems[k])
            cp_pos.wait()
            for k in range(nt):
                cps[k].wait()
                accum(gbufs[k % nbuf])
                if k + nbuf < nt:
                    tbl, iv = tables[k + nbuf]
                    cps[k + nbuf] = pltpu.async_copy(
                        tbl.at[iv.at[sl]], gbufs[(k + nbuf) % nbuf],
                        sems[(k + nbuf) % nbuf])
            dst0 = pl.multiple_of(base + tok0, CHUNK)
            pltpu.sync_copy(acc, out_h.at[pl.ds(dst0, CHUNK)])
            return 0

        lax.fori_loop(0, n_steps, step, 0)

    return gather_sum(ids, x0, y1, x2, y3, word_emb, x_emb, y_emb, h_emb,
                      w_emb, pos_tt)


def _ln_body(x_ref, w_ref, b_ref, o_ref):
    x = x_ref[...]
    mean = jnp.mean(x, axis=-1, keepdims=True)
    xc = x - mean
    var = jnp.mean(xc * xc, axis=-1, keepdims=True)
    o_ref[...] = xc * lax.rsqrt(var + EPS) * w_ref[...] + b_ref[...]


def _layer_norm_tc(x, w, b):
    n_tok = x.shape[0]
    rows = 512
    grid = (n_tok // rows,)
    return pl.pallas_call(
        _ln_body,
        grid=grid,
        in_specs=[
            pl.BlockSpec((rows, HIDDEN), lambda i: (i, 0)),
            pl.BlockSpec((1, HIDDEN), lambda i: (0, 0)),
            pl.BlockSpec((1, HIDDEN), lambda i: (0, 0)),
        ],
        out_specs=pl.BlockSpec((rows, HIDDEN), lambda i: (i, 0)),
        out_shape=jax.ShapeDtypeStruct((n_tok, HIDDEN), jnp.float32),
    )(x, w.reshape(1, HIDDEN), b.reshape(1, HIDDEN))


def kernel(input_ids, bbox, word_emb, pos_emb, x_emb, y_emb, h_emb, w_emb,
           tt_emb, ln_w, ln_b):
    batch, seq = input_ids.shape
    n_tok = batch * seq
    ids = input_ids.reshape(n_tok).astype(jnp.int32)
    bb = bbox.reshape(n_tok, 4).astype(jnp.int32)
    x0, y1, x2, y3 = bb[:, 0], bb[:, 1], bb[:, 2], bb[:, 3]
    pos_tt = pos_emb + tt_emb[0][None, :]
    summed = _sc_gather_sum(ids, x0, y1, x2, y3, word_emb, x_emb, y_emb,
                            h_emb, w_emb, pos_tt)
    out = _layer_norm_tc(summed, ln_w, ln_b)
    return out.reshape(batch, seq, HIDDEN)


# speedup vs baseline: 2.6963x; 1.1895x over previous
import functools

import jax
import jax.numpy as jnp
from jax import lax
from jax.experimental import pallas as pl
from jax.experimental.pallas import tpu as pltpu
from jax.experimental.pallas import tpu_sc as plsc

HIDDEN = 768
EPS = 1e-12
SEQ = 512
CHUNK = 8
NSL = HIDDEN // 16
NTBL = 7


def _sc_gather_sum(ids, x0, y1, x2, y3, word_emb, x_emb, y_emb, h_emb,
                   w_emb):
    n_tok = ids.shape[0]
    info = plsc.get_sparse_core_info()
    n_workers = info.num_cores * info.num_subcores
    per_w = n_tok // n_workers
    n_steps = per_w // CHUNK
    n_half = n_steps // 2

    mesh = plsc.VectorSubcoreMesh(core_axis_name="c", subcore_axis_name="s")

    @functools.partial(
        pl.kernel,
        mesh=mesh,
        out_type=jax.ShapeDtypeStruct((n_tok, HIDDEN), jnp.float32),
        scratch_types=[
            pltpu.VMEM((per_w,), jnp.int32),
            pltpu.VMEM((per_w,), jnp.int32),
            pltpu.VMEM((per_w,), jnp.int32),
            pltpu.VMEM((per_w,), jnp.int32),
            pltpu.VMEM((per_w,), jnp.int32),
            pltpu.VMEM((per_w,), jnp.int32),
            pltpu.VMEM((per_w,), jnp.int32),
            pltpu.VMEM((2, NTBL, CHUNK, HIDDEN), jnp.float32),
            pltpu.VMEM((2, CHUNK, HIDDEN), jnp.float32),
            pltpu.SemaphoreType.DMA,
            pltpu.SemaphoreType.DMA,
            pltpu.SemaphoreType.DMA,
            pltpu.SemaphoreType.DMA,
        ],
    )
    def gather_sum(ids_h, x0_h, y1_h, x2_h, y3_h, word_h, x_h, y_h, h_h,
                   w_h, out_h, ids_v, x0_v, y1_v, x2_v, y3_v, hx_v, wx_v,
                   gset, obuf, gsem0, gsem1, wsem0, wsem1):
        wid = lax.axis_index("s") * info.num_cores + lax.axis_index("c")
        base = wid * per_w

        pltpu.sync_copy(ids_h.at[pl.ds(base, per_w)], ids_v)
        pltpu.sync_copy(x0_h.at[pl.ds(base, per_w)], x0_v)
        pltpu.sync_copy(y1_h.at[pl.ds(base, per_w)], y1_v)
        pltpu.sync_copy(x2_h.at[pl.ds(base, per_w)], x2_v)
        pltpu.sync_copy(y3_h.at[pl.ds(base, per_w)], y3_v)

        def hw_body(j, _):
            s = pl.ds(j * 16, 16)
            hx_v[s] = y3_v[s] - y1_v[s]
            wx_v[s] = x2_v[s] - x0_v[s]
            return 0

        lax.fori_loop(0, per_w // 16, hw_body, 0)

        tables = (
            (word_h, ids_v),
            (x_h, x0_v),
            (x_h, x2_v),
            (y_h, y1_v),
            (y_h, y3_v),
            (h_h, hx_v),
            (w_h, wx_v),
        )
        gsems = (gsem0, gsem1)
        wsems = (wsem0, wsem1)

        def fire_gathers(q, s):
            sl = pl.ds(pl.multiple_of(s * CHUNK, CHUNK), CHUNK)
            for k in range(NTBL):
                tbl, iv = tables[k]
                pltpu.async_copy(tbl.at[iv.at[sl]], gset.at[q, k], gsems[q])

        def drain_gathers(q):
            dummy = word_h.at[pl.ds(0, CHUNK)]
            for k in range(NTBL):
                pltpu.make_async_copy(dummy, gset.at[q, k], gsems[q]).wait()

        def wait_wb(q):
            pltpu.make_async_copy(obuf.at[q], out_h.at[pl.ds(0, CHUNK)],
                                  wsems[q]).wait()

        def accum(q):
            def rbody(r, _):
                for c in range(NSL):
                    cs = pl.ds(c * 16, 16)
                    v = gset[q, 0, r, cs]
                    for k in range(1, NTBL):
                        v = v + gset[q, k, r, cs]
                    obuf[q, r, cs] = v
                return 0

            lax.fori_loop(0, CHUNK, rbody, 0)

        def fire_wb(q, s):
            dst0 = pl.multiple_of(base + s * CHUNK, CHUNK)
            pltpu.async_copy(obuf.at[q], out_h.at[pl.ds(dst0, CHUNK)],
                             wsems[q])

        fire_gathers(0, 0)

        def gbody(g, _):
            s0 = 2 * g
            drain_gathers(0)
            fire_gathers(1, s0 + 1)
            pl.when(g > 0)(lambda: wait_wb(0))
            accum(0)
            fire_wb(0, s0)

            drain_gathers(1)
            pl.when(g < n_half - 1)(lambda: fire_gathers(0, s0 + 2))
            pl.when(g > 0)(lambda: wait_wb(1))
            accum(1)
            fire_wb(1, s0 + 1)
            return 0

        lax.fori_loop(0, n_half, gbody, 0)
        wait_wb(0)
        wait_wb(1)

    return gather_sum(ids, x0, y1, x2, y3, word_emb, x_emb, y_emb, h_emb,
                      w_emb)


def _ln_body(x_ref, p_ref, w_ref, b_ref, o_ref):
    x = x_ref[...] + p_ref[...]
    mean = jnp.mean(x, axis=-1, keepdims=True)
    xc = x - mean
    var = jnp.mean(xc * xc, axis=-1, keepdims=True)
    o_ref[...] = xc * lax.rsqrt(var + EPS) * w_ref[...] + b_ref[...]


def _layer_norm_tc(x, pos_tt, w, b):
    n_tok = x.shape[0]
    rows = SEQ
    grid = (n_tok // rows,)
    return pl.pallas_call(
        _ln_body,
        grid=grid,
        in_specs=[
            pl.BlockSpec((rows, HIDDEN), lambda i: (i, 0)),
            pl.BlockSpec((rows, HIDDEN), lambda i: (0, 0)),
            pl.BlockSpec((1, HIDDEN), lambda i: (0, 0)),
            pl.BlockSpec((1, HIDDEN), lambda i: (0, 0)),
        ],
        out_specs=pl.BlockSpec((rows, HIDDEN), lambda i: (i, 0)),
        out_shape=jax.ShapeDtypeStruct((n_tok, HIDDEN), jnp.float32),
    )(x, pos_tt, w.reshape(1, HIDDEN), b.reshape(1, HIDDEN))


def kernel(input_ids, bbox, word_emb, pos_emb, x_emb, y_emb, h_emb, w_emb,
           tt_emb, ln_w, ln_b):
    batch, seq = input_ids.shape
    n_tok = batch * seq
    ids = input_ids.reshape(n_tok).astype(jnp.int32)
    bb = bbox.reshape(n_tok, 4).astype(jnp.int32)
    x0, y1, x2, y3 = bb[:, 0], bb[:, 1], bb[:, 2], bb[:, 3]
    pos_tt = pos_emb + tt_emb[0][None, :]
    summed = _sc_gather_sum(ids, x0, y1, x2, y3, word_emb, x_emb, y_emb,
                            h_emb, w_emb)
    out = _layer_norm_tc(summed, pos_tt, ln_w, ln_b)
    return out.reshape(batch, seq, HIDDEN)
